# baseline (device time: 948125 ns/iter reference)
import jax
import jax.numpy as jnp
from jax import lax
from jax.experimental import pallas as pl
from jax.experimental.pallas import tpu as pltpu

N_DEV = 32


def kernel(x, w_mat):
    m_per, k = x.shape
    _, n_per = w_mat.shape

    def body(x_ref, w_ref, out_ref, comm_ref, send_sems, recv_sems, credit_sem):
        my = lax.axis_index("i")
        left = lax.rem(my + (N_DEV - 1), N_DEV)
        right = lax.rem(my + 1, N_DEV)

        barrier_sem = pltpu.get_barrier_semaphore()
        for nbr in (left, right):
            pl.semaphore_signal(
                barrier_sem, inc=1,
                device_id=(nbr,), device_id_type=pl.DeviceIdType.MESH,
            )
        pl.semaphore_wait(barrier_sem, 2)

        comm_ref[0] = x_ref[...]
        out_ref[pl.ds(my * m_per, m_per), :] = jnp.dot(
            x_ref[...], w_ref[...], preferred_element_type=jnp.float32
        )

        for h in range(N_DEV - 1):
            send_slot = h % 2
            recv_slot = (h + 1) % 2
            if h >= 1:
                pl.semaphore_wait(credit_sem, 1)
            rdma = pltpu.make_async_remote_copy(
                src_ref=comm_ref.at[send_slot],
                dst_ref=comm_ref.at[recv_slot],
                send_sem=send_sems.at[send_slot],
                recv_sem=recv_sems.at[recv_slot],
                device_id=(right,),
                device_id_type=pl.DeviceIdType.MESH,
            )
            rdma.start()
            rdma.wait()

            origin = lax.rem(my + (N_DEV - 1 - h), N_DEV)
            out_ref[pl.ds(origin * m_per, m_per), :] = jnp.dot(
                comm_ref[recv_slot], w_ref[...], preferred_element_type=jnp.float32
            )
            if h < N_DEV - 2:
                pl.semaphore_signal(
                    credit_sem, inc=1,
                    device_id=(left,), device_id_type=pl.DeviceIdType.MESH,
                )

    out_shape = jax.ShapeDtypeStruct((N_DEV * m_per, n_per), jnp.float32)
    return pl.pallas_call(
        body,
        out_shape=out_shape,
        in_specs=[
            pl.BlockSpec(memory_space=pltpu.VMEM),
            pl.BlockSpec(memory_space=pltpu.VMEM),
        ],
        out_specs=pl.BlockSpec(memory_space=pltpu.VMEM),
        scratch_shapes=[
            pltpu.VMEM((2, m_per, k), jnp.float32),
            pltpu.SemaphoreType.DMA((2,)),
            pltpu.SemaphoreType.DMA((2,)),
            pltpu.SemaphoreType.REGULAR,
        ],
        compiler_params=pltpu.CompilerParams(collective_id=0),
    )(x, w_mat)


# device time: 713655 ns/iter; 1.3285x vs baseline; 1.3285x over previous
import jax
import jax.numpy as jnp
from jax import lax
from jax.experimental import pallas as pl
from jax.experimental.pallas import tpu as pltpu

N_DEV = 32
CW_HOPS = N_DEV // 2
CCW_HOPS = N_DEV // 2 - 1
NSLOT = 3


def kernel(x, w_mat):
    m_per, k = x.shape
    _, n_per = w_mat.shape

    def body(
        x_ref, w_ref, out_ref,
        cw_ref, ccw_ref,
        cw_send_sems, cw_recv_sems, ccw_send_sems, ccw_recv_sems,
        cw_credit, ccw_credit,
    ):
        my = lax.axis_index("i")
        left = lax.rem(my + (N_DEV - 1), N_DEV)
        right = lax.rem(my + 1, N_DEV)

        barrier_sem = pltpu.get_barrier_semaphore()
        for nbr in (left, right):
            pl.semaphore_signal(
                barrier_sem, inc=1,
                device_id=(nbr,), device_id_type=pl.DeviceIdType.MESH,
            )
        pl.semaphore_wait(barrier_sem, 2)

        def copy(src, dst, send_sem, recv_sem, target):
            return pltpu.make_async_remote_copy(
                src_ref=src, dst_ref=dst, send_sem=send_sem,
                recv_sem=recv_sem, device_id=(target,),
                device_id_type=pl.DeviceIdType.MESH,
            )

        send_cw = copy(x_ref, cw_ref.at[0], cw_send_sems.at[0],
                       cw_recv_sems.at[0], right)
        send_cw.start()
        send_ccw = copy(x_ref, ccw_ref.at[0], ccw_send_sems.at[0],
                        ccw_recv_sems.at[0], left)
        send_ccw.start()

        out_ref[pl.ds(my * m_per, m_per), :] = jnp.dot(
            x_ref[...], w_ref[...], preferred_element_type=jnp.float32
        )

        prev_send_cw = send_cw
        prev_send_ccw = send_ccw

        for s in range(CW_HOPS):
            slot = s % NSLOT
            nxt = (s + 1) % NSLOT
            in_ccw = s < CCW_HOPS
            fwd_cw = s + 1 < CW_HOPS
            fwd_ccw = s + 1 < CCW_HOPS

            recv_cw = copy(cw_ref.at[slot], cw_ref.at[slot],
                           cw_send_sems.at[slot], cw_recv_sems.at[slot], right)
            recv_cw.wait_recv()
            next_send_cw = None
            if fwd_cw:
                if s + 1 >= NSLOT:
                    pl.semaphore_wait(cw_credit, 1)
                next_send_cw = copy(cw_ref.at[slot], cw_ref.at[nxt],
                                    cw_send_sems.at[nxt], cw_recv_sems.at[nxt],
                                    right)
                next_send_cw.start()

            next_send_ccw = None
            if in_ccw:
                recv_ccw = copy(ccw_ref.at[slot], ccw_ref.at[slot],
                                ccw_send_sems.at[slot], ccw_recv_sems.at[slot],
                                left)
                recv_ccw.wait_recv()
                if fwd_ccw:
                    if s + 1 >= NSLOT:
                        pl.semaphore_wait(ccw_credit, 1)
                    next_send_ccw = copy(ccw_ref.at[slot], ccw_ref.at[nxt],
                                         ccw_send_sems.at[nxt],
                                         ccw_recv_sems.at[nxt], left)
                    next_send_ccw.start()

            prev_send_cw.wait_send()
            if 1 <= s <= CW_HOPS - 3:
                pl.semaphore_signal(
                    cw_credit, inc=1,
                    device_id=(left,), device_id_type=pl.DeviceIdType.MESH,
                )
            if in_ccw:
                prev_send_ccw.wait_send()
                if 1 <= s <= CCW_HOPS - 3:
                    pl.semaphore_signal(
                        ccw_credit, inc=1,
                        device_id=(right,), device_id_type=pl.DeviceIdType.MESH,
                    )

            origin_cw = lax.rem(my + (N_DEV - 1 - s), N_DEV)
            out_ref[pl.ds(origin_cw * m_per, m_per), :] = jnp.dot(
                cw_ref[slot], w_ref[...], preferred_element_type=jnp.float32
            )
            if in_ccw:
                origin_ccw = lax.rem(my + 1 + s, N_DEV)
                out_ref[pl.ds(origin_ccw * m_per, m_per), :] = jnp.dot(
                    ccw_ref[slot], w_ref[...],
                    preferred_element_type=jnp.float32
                )

            prev_send_cw = next_send_cw
            prev_send_ccw = next_send_ccw

    out_shape = jax.ShapeDtypeStruct((N_DEV * m_per, n_per), jnp.float32)
    return pl.pallas_call(
        body,
        out_shape=out_shape,
        in_specs=[
            pl.BlockSpec(memory_space=pltpu.VMEM),
            pl.BlockSpec(memory_space=pltpu.VMEM),
        ],
        out_specs=pl.BlockSpec(memory_space=pltpu.VMEM),
        scratch_shapes=[
            pltpu.VMEM((NSLOT, m_per, k), jnp.float32),
            pltpu.VMEM((NSLOT, m_per, k), jnp.float32),
            pltpu.SemaphoreType.DMA((NSLOT,)),
            pltpu.SemaphoreType.DMA((NSLOT,)),
            pltpu.SemaphoreType.DMA((NSLOT,)),
            pltpu.SemaphoreType.DMA((NSLOT,)),
            pltpu.SemaphoreType.REGULAR,
            pltpu.SemaphoreType.REGULAR,
        ],
        compiler_params=pltpu.CompilerParams(collective_id=0),
    )(x, w_mat)
